# baseline (speedup 1.0000x reference)
import functools

import jax
import jax.numpy as jnp
from jax import lax
from jax.experimental import pallas as pl
from jax.experimental.pallas import tpu as pltpu
from jax.experimental.pallas import tpu_sc as plsc

NC = 2
NS = 16
NW = NC * NS
CH = 100


def _deg_mesh_kernel(N, rows_per_tile):
    mesh = plsc.VectorSubcoreMesh(core_axis_name="c", subcore_axis_name="s")
    n16 = N // 16
    copiers = N // 1000

    @functools.partial(
        pl.kernel,
        mesh=mesh,
        compiler_params=pltpu.CompilerParams(use_tc_tiling_on_sc=False),
        out_type=jax.ShapeDtypeStruct((NC * N,), jnp.float32),
        scratch_types=[
            pltpu.VMEM((rows_per_tile, CH), jnp.int32),
            pltpu.VMEM((CH,), jnp.float32),
            pltpu.VMEM((N,), jnp.float32),
            pltpu.VMEM_SHARED((N,), jnp.float32),
        ],
    )
    def deg_kernel(ei_hbm, out_hbm, idx_v, ones_v, zbuf_v, acc_sh):
        c = lax.axis_index("c")
        s = lax.axis_index("s")

        pltpu.sync_copy(
            ei_hbm.at[1, s, pl.ds(c * rows_per_tile, rows_per_tile)], idx_v)

        for j in range(CH // 16):
            ones_v[pl.ds(j * 16, 16)] = jnp.ones((16,), jnp.float32)

        @pl.when(s == 0)
        def _():
            def zb(i, carry):
                zbuf_v[pl.ds(i * 16, 16)] = jnp.zeros((16,), jnp.float32)
                return carry
            lax.fori_loop(0, n16, zb, None)
            pltpu.sync_copy(zbuf_v, acc_sh)

        plsc.subcore_barrier()

        def body(i, carry):
            pltpu.sync_copy(ones_v, acc_sh.at[idx_v.at[i]], add=True)
            return carry
        lax.fori_loop(0, rows_per_tile, body, None)

        plsc.subcore_barrier()

        @pl.when(s < copiers)
        def _():
            pltpu.sync_copy(acc_sh.at[pl.ds(s * 1000, 1000)],
                            zbuf_v.at[pl.ds(0, 1000)])
            pltpu.sync_copy(zbuf_v.at[pl.ds(0, 1000)],
                            out_hbm.at[pl.ds(c * N + s * 1000, 1000)])

    return deg_kernel


def _agg_mesh_kernel(N, C2, rows_per_tile):
    mesh = plsc.VectorSubcoreMesh(core_axis_name="c", subcore_axis_name="s")
    NB = 6
    PF = NB // 2
    zrows = 500
    copiers = N // 1000

    @functools.partial(
        pl.kernel,
        mesh=mesh,
        compiler_params=pltpu.CompilerParams(use_tc_tiling_on_sc=False),
        out_type=jax.ShapeDtypeStruct((NC, N, C2), jnp.float32),
        scratch_types=[
            pltpu.VMEM((rows_per_tile, CH), jnp.int32),
            pltpu.VMEM((rows_per_tile, CH), jnp.int32),
            pltpu.VMEM((NB * CH, C2), jnp.float32),
            pltpu.VMEM_SHARED((N, C2), jnp.float32),
            pltpu.SemaphoreType.DMA((NB,)),
            pltpu.SemaphoreType.DMA((NB,)),
        ],
    )
    def agg_kernel(ys_hbm, ei_hbm, out_hbm,
                   src_v, dst_v, rows_v, acc_sh, gsem, ssem):
        c = lax.axis_index("c")
        s = lax.axis_index("s")
        y_c = ys_hbm.at[c]
        out_c = out_hbm.at[c]

        def bankref(b):
            return rows_v.at[pl.ds(b * CH, CH)]

        pltpu.sync_copy(ei_hbm.at[0, s], src_v)
        pltpu.sync_copy(ei_hbm.at[1, s], dst_v)

        def zb(i, carry):
            for j in range(C2 // 16):
                rows_v[i, pl.ds(j * 16, 16)] = jnp.zeros((16,), jnp.float32)
            return carry
        lax.fori_loop(0, zrows, zb, None)

        @pl.when(s < copiers)
        def _():
            for k in range(1000 // zrows):
                pltpu.sync_copy(rows_v.at[pl.ds(0, zrows)],
                                acc_sh.at[pl.ds(s * 1000 + k * zrows, zrows)])

        plsc.subcore_barrier()

        for k in range(PF):
            pltpu.async_copy(y_c.at[src_v.at[k]], bankref(k), gsem.at[k])

        def body(i, carry):
            bank = lax.rem(i, NB)
            fbank = lax.rem(i + PF, NB)

            @pl.when(i >= PF)
            def _():
                pltpu.make_async_copy(bankref(fbank),
                                      acc_sh.at[dst_v.at[i - PF]],
                                      ssem.at[fbank]).wait()

            @pl.when(i + PF < rows_per_tile)
            def _():
                pltpu.async_copy(y_c.at[src_v.at[i + PF]],
                                 bankref(fbank), gsem.at[fbank])

            pltpu.make_async_copy(y_c.at[src_v.at[i]], bankref(bank),
                                  gsem.at[bank]).wait()

            pltpu.async_copy(bankref(bank), acc_sh.at[dst_v.at[i]],
                             ssem.at[bank], add=True)
            return carry
        lax.fori_loop(0, rows_per_tile, body, None)

        for k in range(PF, 0, -1):
            last = rows_per_tile - k
            pltpu.make_async_copy(bankref(lax.rem(last, NB)),
                                  acc_sh.at[dst_v.at[last]],
                                  ssem.at[lax.rem(last, NB)]).wait()

        plsc.subcore_barrier()

        @pl.when(s < copiers)
        def _():
            for k in range(1000 // zrows):
                pltpu.sync_copy(acc_sh.at[pl.ds(s * 1000 + k * zrows, zrows)],
                                rows_v.at[pl.ds(0, zrows)])
                pltpu.sync_copy(rows_v.at[pl.ds(0, zrows)],
                                out_c.at[pl.ds(s * 1000 + k * zrows, zrows)])

    return agg_kernel


def _matmul_tc_kernel(x_ref, w_ref, xw_ref):
    xw_ref[...] = jnp.dot(x_ref[...], w_ref[...],
                          preferred_element_type=jnp.float32)


def _scale_tc_kernel(xw_ref, degt_ref, ys_ref):
    C2 = ys_ref.shape[2]
    deg = degt_ref[:, 0:1] + degt_ref[:, 1:2] + 1.0
    dis = lax.rsqrt(deg)
    y = xw_ref[...] * dis
    ys_ref[0] = y[:, :C2]
    ys_ref[1] = y[:, C2:]


def _final_tc_kernel(agg_ref, ys_ref, degt_ref, b_ref, out_ref):
    deg = degt_ref[:, 0:1] + degt_ref[:, 1:2] + 1.0
    dis = lax.rsqrt(deg)
    z = jnp.concatenate([agg_ref[0] + ys_ref[0], agg_ref[1] + ys_ref[1]],
                        axis=1) * dis
    out_ref[...] = z + b_ref[...]


def kernel(x, edge_index, W, b):
    N, C = x.shape
    C2 = C // 2
    E = edge_index.shape[1]
    assert E % (NW * CH) == 0 and N % 1000 == 0 and C % 32 == 0
    deg_rows_per_tile = E // CH // NW
    agg_rows_per_tile = E // CH // NS

    ei3 = edge_index.astype(jnp.int32).reshape(2, NS, agg_rows_per_tile, CH)

    blk = 1000
    grid = (N // blk,)

    xw = pl.pallas_call(
        _matmul_tc_kernel,
        grid=grid,
        in_specs=[
            pl.BlockSpec((blk, C), lambda i: (i, 0)),
            pl.BlockSpec((C, C), lambda i: (0, 0)),
        ],
        out_specs=pl.BlockSpec((blk, C), lambda i: (i, 0)),
        out_shape=jax.ShapeDtypeStruct((N, C), jnp.float32),
    )(x, W)

    degp = _deg_mesh_kernel(N, deg_rows_per_tile)(ei3).reshape(NC, N)
    degt = jnp.transpose(degp)

    ys = pl.pallas_call(
        _scale_tc_kernel,
        grid=grid,
        in_specs=[
            pl.BlockSpec((blk, C), lambda i: (i, 0)),
            pl.BlockSpec((blk, NC), lambda i: (i, 0)),
        ],
        out_specs=pl.BlockSpec((NC, blk, C2), lambda i: (0, i, 0)),
        out_shape=jax.ShapeDtypeStruct((NC, N, C2), jnp.float32),
    )(xw, degt)

    agg = _agg_mesh_kernel(N, C2, agg_rows_per_tile)(ys, ei3)

    out = pl.pallas_call(
        _final_tc_kernel,
        grid=grid,
        in_specs=[
            pl.BlockSpec((NC, blk, C2), lambda i: (0, i, 0)),
            pl.BlockSpec((NC, blk, C2), lambda i: (0, i, 0)),
            pl.BlockSpec((blk, NC), lambda i: (i, 0)),
            pl.BlockSpec((1, C), lambda i: (0, 0)),
        ],
        out_specs=pl.BlockSpec((blk, C), lambda i: (i, 0)),
        out_shape=jax.ShapeDtypeStruct((N, C), jnp.float32),
    )(agg, ys, degt, b.reshape(1, C))
    return out


# speedup vs baseline: 1.1041x; 1.1041x over previous
import functools

import jax
import jax.numpy as jnp
from jax import lax
from jax.experimental import pallas as pl
from jax.experimental.pallas import tpu as pltpu
from jax.experimental.pallas import tpu_sc as plsc

NC = 2
NS = 16
NW = NC * NS
CH = 80


def _deg_mesh_kernel(N, rows_per_tile):
    mesh = plsc.VectorSubcoreMesh(core_axis_name="c", subcore_axis_name="s")
    n16 = N // 16
    copiers = N // 1000

    @functools.partial(
        pl.kernel,
        mesh=mesh,
        compiler_params=pltpu.CompilerParams(use_tc_tiling_on_sc=False),
        out_type=jax.ShapeDtypeStruct((NC * N,), jnp.float32),
        scratch_types=[
            pltpu.VMEM((rows_per_tile, CH), jnp.int32),
            pltpu.VMEM((CH,), jnp.float32),
            pltpu.VMEM((N,), jnp.float32),
            pltpu.VMEM_SHARED((N,), jnp.float32),
            pltpu.SemaphoreType.DMA,
        ],
    )
    def deg_kernel(ei_hbm, out_hbm, idx_v, ones_v, zbuf_v, acc_sh, dsem):
        c = lax.axis_index("c")
        s = lax.axis_index("s")

        pltpu.sync_copy(
            ei_hbm.at[1, s, pl.ds(c * rows_per_tile, rows_per_tile)], idx_v)

        for j in range(CH // 16):
            ones_v[pl.ds(j * 16, 16)] = jnp.ones((16,), jnp.float32)

        @pl.when(s == 0)
        def _():
            def zb(i, carry):
                zbuf_v[pl.ds(i * 16, 16)] = jnp.zeros((16,), jnp.float32)
                return carry
            lax.fori_loop(0, n16, zb, None)
            pltpu.sync_copy(zbuf_v, acc_sh)

        plsc.subcore_barrier()

        def body(i, carry):
            pltpu.async_copy(ones_v, acc_sh.at[idx_v.at[i]], dsem, add=True)
            return carry
        lax.fori_loop(0, rows_per_tile, body, None)

        def drain(i, carry):
            pltpu.make_async_copy(ones_v, acc_sh.at[idx_v.at[0]], dsem).wait()
            return carry
        lax.fori_loop(0, rows_per_tile, drain, None)

        plsc.subcore_barrier()

        @pl.when(s < copiers)
        def _():
            pltpu.sync_copy(acc_sh.at[pl.ds(s * 1000, 1000)],
                            zbuf_v.at[pl.ds(0, 1000)])
            pltpu.sync_copy(zbuf_v.at[pl.ds(0, 1000)],
                            out_hbm.at[pl.ds(c * N + s * 1000, 1000)])

    return deg_kernel


def _agg_mesh_kernel(N, C2, rows_per_tile):
    mesh = plsc.VectorSubcoreMesh(core_axis_name="c", subcore_axis_name="s")
    NB = 6
    PF = NB // 2
    zrows = 500
    copiers = N // 1000

    @functools.partial(
        pl.kernel,
        mesh=mesh,
        compiler_params=pltpu.CompilerParams(use_tc_tiling_on_sc=False),
        out_type=jax.ShapeDtypeStruct((NC, N, C2), jnp.float32),
        scratch_types=[
            pltpu.VMEM((rows_per_tile, CH), jnp.int32),
            pltpu.VMEM((rows_per_tile, CH), jnp.int32),
            pltpu.VMEM((NB * CH, C2), jnp.float32),
            pltpu.VMEM_SHARED((N, C2), jnp.float32),
            pltpu.SemaphoreType.DMA((NB,)),
            pltpu.SemaphoreType.DMA((NB,)),
        ],
    )
    def agg_kernel(ys_hbm, ei_hbm, out_hbm,
                   src_v, dst_v, rows_v, acc_sh, gsem, ssem):
        c = lax.axis_index("c")
        s = lax.axis_index("s")
        y_c = ys_hbm.at[c]
        out_c = out_hbm.at[c]

        def bankref(b):
            return rows_v.at[pl.ds(b * CH, CH)]

        pltpu.sync_copy(ei_hbm.at[0, s], src_v)
        pltpu.sync_copy(ei_hbm.at[1, s], dst_v)

        def zb(i, carry):
            for j in range(C2 // 16):
                rows_v[i, pl.ds(j * 16, 16)] = jnp.zeros((16,), jnp.float32)
            return carry
        lax.fori_loop(0, zrows, zb, None)

        @pl.when(s < copiers)
        def _():
            for k in range(1000 // zrows):
                pltpu.sync_copy(rows_v.at[pl.ds(0, zrows)],
                                acc_sh.at[pl.ds(s * 1000 + k * zrows, zrows)])

        plsc.subcore_barrier()

        for k in range(PF):
            pltpu.async_copy(y_c.at[src_v.at[k]], bankref(k), gsem.at[k])

        def body(i, carry):
            bank = lax.rem(i, NB)
            fbank = lax.rem(i + PF, NB)

            @pl.when(i >= PF)
            def _():
                pltpu.make_async_copy(bankref(fbank),
                                      acc_sh.at[dst_v.at[i - PF]],
                                      ssem.at[fbank]).wait()

            @pl.when(i + PF < rows_per_tile)
            def _():
                pltpu.async_copy(y_c.at[src_v.at[i + PF]],
                                 bankref(fbank), gsem.at[fbank])

            pltpu.make_async_copy(y_c.at[src_v.at[i]], bankref(bank),
                                  gsem.at[bank]).wait()

            pltpu.async_copy(bankref(bank), acc_sh.at[dst_v.at[i]],
                             ssem.at[bank], add=True)
            return carry
        lax.fori_loop(0, rows_per_tile, body, None)

        for k in range(PF, 0, -1):
            last = rows_per_tile - k
            pltpu.make_async_copy(bankref(lax.rem(last, NB)),
                                  acc_sh.at[dst_v.at[last]],
                                  ssem.at[lax.rem(last, NB)]).wait()

        plsc.subcore_barrier()

        @pl.when(s < copiers)
        def _():
            for k in range(1000 // zrows):
                pltpu.sync_copy(acc_sh.at[pl.ds(s * 1000 + k * zrows, zrows)],
                                rows_v.at[pl.ds(0, zrows)])
                pltpu.sync_copy(rows_v.at[pl.ds(0, zrows)],
                                out_c.at[pl.ds(s * 1000 + k * zrows, zrows)])

    return agg_kernel


def _matmul_tc_kernel(x_ref, w_ref, xw_ref):
    xw_ref[...] = jnp.dot(x_ref[...], w_ref[...],
                          preferred_element_type=jnp.float32)


def _scale_tc_kernel(xw_ref, degt_ref, ys_ref):
    C2 = ys_ref.shape[2]
    deg = degt_ref[:, 0:1] + degt_ref[:, 1:2] + 1.0
    dis = lax.rsqrt(deg)
    y = xw_ref[...] * dis
    ys_ref[0] = y[:, :C2]
    ys_ref[1] = y[:, C2:]


def _final_tc_kernel(agg_ref, ys_ref, degt_ref, b_ref, out_ref):
    deg = degt_ref[:, 0:1] + degt_ref[:, 1:2] + 1.0
    dis = lax.rsqrt(deg)
    z = jnp.concatenate([agg_ref[0] + ys_ref[0], agg_ref[1] + ys_ref[1]],
                        axis=1) * dis
    out_ref[...] = z + b_ref[...]


def kernel(x, edge_index, W, b):
    N, C = x.shape
    C2 = C // 2
    E = edge_index.shape[1]
    assert E % (NW * CH) == 0 and N % 1000 == 0 and C % 32 == 0
    deg_rows_per_tile = E // CH // NW
    agg_rows_per_tile = E // CH // NS

    ei3 = edge_index.astype(jnp.int32).reshape(2, NS, agg_rows_per_tile, CH)

    blk = 1000
    grid = (N // blk,)

    xw = pl.pallas_call(
        _matmul_tc_kernel,
        grid=grid,
        in_specs=[
            pl.BlockSpec((blk, C), lambda i: (i, 0)),
            pl.BlockSpec((C, C), lambda i: (0, 0)),
        ],
        out_specs=pl.BlockSpec((blk, C), lambda i: (i, 0)),
        out_shape=jax.ShapeDtypeStruct((N, C), jnp.float32),
    )(x, W)

    degp = _deg_mesh_kernel(N, deg_rows_per_tile)(ei3).reshape(NC, N)
    degt = jnp.transpose(degp)

    ys = pl.pallas_call(
        _scale_tc_kernel,
        grid=grid,
        in_specs=[
            pl.BlockSpec((blk, C), lambda i: (i, 0)),
            pl.BlockSpec((blk, NC), lambda i: (i, 0)),
        ],
        out_specs=pl.BlockSpec((NC, blk, C2), lambda i: (0, i, 0)),
        out_shape=jax.ShapeDtypeStruct((NC, N, C2), jnp.float32),
    )(xw, degt)

    agg = _agg_mesh_kernel(N, C2, agg_rows_per_tile)(ys, ei3)

    out = pl.pallas_call(
        _final_tc_kernel,
        grid=grid,
        in_specs=[
            pl.BlockSpec((NC, blk, C2), lambda i: (0, i, 0)),
            pl.BlockSpec((NC, blk, C2), lambda i: (0, i, 0)),
            pl.BlockSpec((blk, NC), lambda i: (i, 0)),
            pl.BlockSpec((1, C), lambda i: (0, 0)),
        ],
        out_specs=pl.BlockSpec((blk, C), lambda i: (i, 0)),
        out_shape=jax.ShapeDtypeStruct((N, C), jnp.float32),
    )(agg, ys, degt, b.reshape(1, C))
    return out


# speedup vs baseline: 1.1080x; 1.0035x over previous
import functools

import jax
import jax.numpy as jnp
from jax import lax
from jax.experimental import pallas as pl
from jax.experimental.pallas import tpu as pltpu
from jax.experimental.pallas import tpu_sc as plsc

NC = 2
NS = 16
NW = NC * NS
CH = 80


def _deg_mesh_kernel(N, rows_per_tile):
    mesh = plsc.VectorSubcoreMesh(core_axis_name="c", subcore_axis_name="s")
    n16 = N // 16
    copiers = N // 1000

    @functools.partial(
        pl.kernel,
        mesh=mesh,
        compiler_params=pltpu.CompilerParams(use_tc_tiling_on_sc=False),
        out_type=jax.ShapeDtypeStruct((NC * N,), jnp.float32),
        scratch_types=[
            pltpu.VMEM((rows_per_tile, CH), jnp.int32),
            pltpu.VMEM((CH,), jnp.float32),
            pltpu.VMEM((N,), jnp.float32),
            pltpu.VMEM_SHARED((N,), jnp.float32),
            pltpu.SemaphoreType.DMA,
        ],
    )
    def deg_kernel(ei_hbm, out_hbm, idx_v, ones_v, zbuf_v, acc_sh, dsem):
        c = lax.axis_index("c")
        s = lax.axis_index("s")

        pltpu.sync_copy(
            ei_hbm.at[1, s, pl.ds(c * rows_per_tile, rows_per_tile)], idx_v)

        for j in range(CH // 16):
            ones_v[pl.ds(j * 16, 16)] = jnp.ones((16,), jnp.float32)

        @pl.when(s == 0)
        def _():
            def zb(i, carry):
                zbuf_v[pl.ds(i * 16, 16)] = jnp.zeros((16,), jnp.float32)
                return carry
            lax.fori_loop(0, n16, zb, None)
            pltpu.sync_copy(zbuf_v, acc_sh)

        plsc.subcore_barrier()

        def body(i, carry):
            pltpu.async_copy(ones_v, acc_sh.at[idx_v.at[i]], dsem, add=True)
            return carry
        lax.fori_loop(0, rows_per_tile, body, None)

        def drain(i, carry):
            pltpu.make_async_copy(ones_v, acc_sh.at[idx_v.at[0]], dsem).wait()
            return carry
        lax.fori_loop(0, rows_per_tile, drain, None)

        plsc.subcore_barrier()

        @pl.when(s < copiers)
        def _():
            pltpu.sync_copy(acc_sh.at[pl.ds(s * 1000, 1000)],
                            zbuf_v.at[pl.ds(0, 1000)])
            pltpu.sync_copy(zbuf_v.at[pl.ds(0, 1000)],
                            out_hbm.at[pl.ds(c * N + s * 1000, 1000)])

    return deg_kernel


def _agg_mesh_kernel(N, C2, rows_per_tile):
    mesh = plsc.VectorSubcoreMesh(core_axis_name="c", subcore_axis_name="s")
    NB = 6
    PF = NB // 2
    zrows = 250
    assert zrows <= NB * CH and 1000 % zrows == 0
    copiers = N // 1000

    @functools.partial(
        pl.kernel,
        mesh=mesh,
        compiler_params=pltpu.CompilerParams(use_tc_tiling_on_sc=False),
        out_type=jax.ShapeDtypeStruct((NC, N, C2), jnp.float32),
        scratch_types=[
            pltpu.VMEM((rows_per_tile, CH), jnp.int32),
            pltpu.VMEM((rows_per_tile, CH), jnp.int32),
            pltpu.VMEM((NB * CH, C2), jnp.float32),
            pltpu.VMEM_SHARED((N, C2), jnp.float32),
            pltpu.SemaphoreType.DMA((NB,)),
            pltpu.SemaphoreType.DMA((NB,)),
        ],
    )
    def agg_kernel(ys_hbm, ei_hbm, out_hbm,
                   src_v, dst_v, rows_v, acc_sh, gsem, ssem):
        c = lax.axis_index("c")
        s = lax.axis_index("s")
        y_c = ys_hbm.at[c]
        out_c = out_hbm.at[c]

        def bankref(b):
            return rows_v.at[pl.ds(b * CH, CH)]

        pltpu.sync_copy(ei_hbm.at[0, s], src_v)
        pltpu.sync_copy(ei_hbm.at[1, s], dst_v)

        def zb(i, carry):
            for j in range(C2 // 16):
                rows_v[i, pl.ds(j * 16, 16)] = jnp.zeros((16,), jnp.float32)
            return carry
        lax.fori_loop(0, zrows, zb, None)

        @pl.when(s < copiers)
        def _():
            for k in range(1000 // zrows):
                pltpu.sync_copy(rows_v.at[pl.ds(0, zrows)],
                                acc_sh.at[pl.ds(s * 1000 + k * zrows, zrows)])

        plsc.subcore_barrier()

        for k in range(PF):
            pltpu.async_copy(y_c.at[src_v.at[k]], bankref(k), gsem.at[k])

        def body(i, carry):
            bank = lax.rem(i, NB)
            fbank = lax.rem(i + PF, NB)

            @pl.when(i >= PF)
            def _():
                pltpu.make_async_copy(bankref(fbank),
                                      acc_sh.at[dst_v.at[i - PF]],
                                      ssem.at[fbank]).wait()

            @pl.when(i + PF < rows_per_tile)
            def _():
                pltpu.async_copy(y_c.at[src_v.at[i + PF]],
                                 bankref(fbank), gsem.at[fbank])

            pltpu.make_async_copy(y_c.at[src_v.at[i]], bankref(bank),
                                  gsem.at[bank]).wait()

            pltpu.async_copy(bankref(bank), acc_sh.at[dst_v.at[i]],
                             ssem.at[bank], add=True)
            return carry
        lax.fori_loop(0, rows_per_tile, body, None)

        for k in range(PF, 0, -1):
            last = rows_per_tile - k
            pltpu.make_async_copy(bankref(lax.rem(last, NB)),
                                  acc_sh.at[dst_v.at[last]],
                                  ssem.at[lax.rem(last, NB)]).wait()

        plsc.subcore_barrier()

        @pl.when(s < copiers)
        def _():
            for k in range(1000 // zrows):
                pltpu.sync_copy(acc_sh.at[pl.ds(s * 1000 + k * zrows, zrows)],
                                rows_v.at[pl.ds(0, zrows)])
                pltpu.sync_copy(rows_v.at[pl.ds(0, zrows)],
                                out_c.at[pl.ds(s * 1000 + k * zrows, zrows)])

    return agg_kernel


def _matmul_tc_kernel(x_ref, w_ref, xw_ref):
    xw_ref[...] = jnp.dot(x_ref[...], w_ref[...],
                          preferred_element_type=jnp.float32)


def _scale_tc_kernel(xw_ref, degt_ref, ys_ref):
    C2 = ys_ref.shape[2]
    deg = degt_ref[:, 0:1] + degt_ref[:, 1:2] + 1.0
    dis = lax.rsqrt(deg)
    y = xw_ref[...] * dis
    ys_ref[0] = y[:, :C2]
    ys_ref[1] = y[:, C2:]


def _final_tc_kernel(agg_ref, ys_ref, degt_ref, b_ref, out_ref):
    deg = degt_ref[:, 0:1] + degt_ref[:, 1:2] + 1.0
    dis = lax.rsqrt(deg)
    z = jnp.concatenate([agg_ref[0] + ys_ref[0], agg_ref[1] + ys_ref[1]],
                        axis=1) * dis
    out_ref[...] = z + b_ref[...]


def kernel(x, edge_index, W, b):
    N, C = x.shape
    C2 = C // 2
    E = edge_index.shape[1]
    assert E % (NW * CH) == 0 and N % 1000 == 0 and C % 32 == 0
    deg_rows_per_tile = E // CH // NW
    agg_rows_per_tile = E // CH // NS

    ei3 = edge_index.astype(jnp.int32).reshape(2, NS, agg_rows_per_tile, CH)

    blk = 1000
    grid = (N // blk,)

    xw = pl.pallas_call(
        _matmul_tc_kernel,
        grid=grid,
        in_specs=[
            pl.BlockSpec((blk, C), lambda i: (i, 0)),
            pl.BlockSpec((C, C), lambda i: (0, 0)),
        ],
        out_specs=pl.BlockSpec((blk, C), lambda i: (i, 0)),
        out_shape=jax.ShapeDtypeStruct((N, C), jnp.float32),
    )(x, W)

    degp = _deg_mesh_kernel(N, deg_rows_per_tile)(ei3).reshape(NC, N)
    degt = jnp.transpose(degp)

    ys = pl.pallas_call(
        _scale_tc_kernel,
        grid=grid,
        in_specs=[
            pl.BlockSpec((blk, C), lambda i: (i, 0)),
            pl.BlockSpec((blk, NC), lambda i: (i, 0)),
        ],
        out_specs=pl.BlockSpec((NC, blk, C2), lambda i: (0, i, 0)),
        out_shape=jax.ShapeDtypeStruct((NC, N, C2), jnp.float32),
    )(xw, degt)

    agg = _agg_mesh_kernel(N, C2, agg_rows_per_tile)(ys, ei3)

    out = pl.pallas_call(
        _final_tc_kernel,
        grid=grid,
        in_specs=[
            pl.BlockSpec((NC, blk, C2), lambda i: (0, i, 0)),
            pl.BlockSpec((NC, blk, C2), lambda i: (0, i, 0)),
            pl.BlockSpec((blk, NC), lambda i: (i, 0)),
            pl.BlockSpec((1, C), lambda i: (0, 0)),
        ],
        out_specs=pl.BlockSpec((blk, C), lambda i: (i, 0)),
        out_shape=jax.ShapeDtypeStruct((N, C), jnp.float32),
    )(agg, ys, degt, b.reshape(1, C))
    return out
